# baseline (speedup 1.0000x reference)
import functools

import jax
import jax.numpy as jnp
import numpy as np
from jax import lax
from jax.experimental import pallas as pl
from jax.experimental.pallas import tpu as pltpu
from jax.experimental.pallas import tpu_sc as plsc

_N = 1000000
_B = 4096
_D = 64
_NC = 2
_NS = 16
_NW = _NC * _NS
_BPW = _B // _NW
_L = 16
_W = _BPW // _L


@functools.partial(
    pl.kernel,
    out_type=[
        jax.ShapeDtypeStruct((_B,), jnp.float32),
        jax.ShapeDtypeStruct((_B,), jnp.float32),
    ],
    mesh=plsc.VectorSubcoreMesh(core_axis_name="c", subcore_axis_name="s"),
    scratch_types=[
        pltpu.VMEM((8, _BPW), jnp.int32),
        pltpu.VMEM((8, _BPW), jnp.int32),
        pltpu.VMEM((_BPW, 2 * _D), jnp.float32),
        pltpu.VMEM((_BPW, 2 * _D), jnp.float32),
        pltpu.VMEM((_BPW, 2 * _D), jnp.float32),
        pltpu.VMEM((_BPW, 2 * _D), jnp.float32),
        pltpu.VMEM((2, _BPW), jnp.float32),
        pltpu.SemaphoreType.DMA,
    ],
)
def _ttranse_sc(gidx_hbm, gpar_hbm, ent_hbm, rel_hbm,
                out_correct, out_corrupt,
                gidx_v, gpar_v, hb, rb, tb, ttb, out_v, sem):
    wid = lax.axis_index("s") * _NC + lax.axis_index("c")
    base = wid * _BPW

    pltpu.sync_copy(gidx_hbm.at[:, pl.ds(base, _BPW)], gidx_v)
    pltpu.sync_copy(gpar_hbm.at[:, pl.ds(base, _BPW)], gpar_v)

    bufs = (hb, rb, tb, ttb)
    tabs = (ent_hbm, rel_hbm, ent_hbm, rel_hbm)

    lane = lax.iota(jnp.int32, _L)
    perms = [lane ^ m for m in (8, 4, 2, 1)]

    def lane_shuffle(v, idx):
        return lax.gather(
            v, idx[:, None],
            lax.GatherDimensionNumbers(
                offset_dims=(), collapsed_slice_dims=(0,),
                start_index_map=(0,)),
            (1,), mode=lax.GatherScatterMode.PROMISE_IN_BOUNDS)

    for b in range(2):
        copies = [
            pltpu.async_copy(tabs[t].at[gidx_v.at[4 * b + t]], bufs[t], sem)
            for t in range(4)
        ]
        for c in copies:
            c.wait()

        def wave(w, carry, b=b):
            sl_w = pl.ds(w * _L, _L)
            pars = [gpar_v[4 * b + t, sl_w].astype(jnp.float32)
                    for t in range(4)]
            vec = jnp.zeros((_L,), jnp.float32)
            for j in range(_L):
                rj = w * _L + j
                bj = lane * 0 + j
                pf = [lane_shuffle(pars[t], bj) for t in range(4)]
                acc = jnp.zeros((_L,), jnp.float32)
                for c in range(_D // _L):
                    vals = []
                    for t in range(4):
                        lo = bufs[t][rj, pl.ds(c * _L, _L)]
                        hi = bufs[t][rj, pl.ds(_D + c * _L, _L)]
                        vals.append(lo + (hi - lo) * pf[t])
                    v = vals[0] + vals[1] + vals[3] - vals[2]
                    acc = acc + v * v
                for pm in perms:
                    acc = acc + lane_shuffle(acc, pm)
                onehot = (1 - jnp.minimum(jnp.abs(lane - j), 1)
                          ).astype(jnp.float32)
                vec = vec + acc * onehot
            out_v[b, sl_w] = vec
            return carry

        lax.fori_loop(0, _W, wave, 0)

    pltpu.sync_copy(out_v.at[0], out_correct.at[pl.ds(base, _BPW)])
    pltpu.sync_copy(out_v.at[1], out_corrupt.at[pl.ds(base, _BPW)])


def kernel(batch, corrupt_batch, entity_emb, relation_emb):
    idx = jnp.concatenate([batch.T, corrupt_batch.T], axis=0)
    ent2 = entity_emb.reshape(_N // 2, 2 * _D)
    rel2 = relation_emb.reshape(_N // 2, 2 * _D)
    gidx = idx >> 1
    gpar = idx & 1
    correct, corrupt = _ttranse_sc(gidx, gpar, ent2, rel2)
    return (correct, corrupt)


# speedup vs baseline: 1.5660x; 1.5660x over previous
import functools

import jax
import jax.numpy as jnp
from jax import lax
from jax.experimental import pallas as pl
from jax.experimental.pallas import tpu as pltpu
from jax.experimental.pallas import tpu_sc as plsc

_N = 1000000
_B = 4096
_D = 64
_NC = 2
_NS = 16
_NW = _NC * _NS
_BPW = _B // _NW
_L = 16
_W = _BPW // _L


@functools.partial(
    pl.kernel,
    out_type=[
        jax.ShapeDtypeStruct((_B,), jnp.float32),
        jax.ShapeDtypeStruct((_B,), jnp.float32),
    ],
    mesh=plsc.VectorSubcoreMesh(core_axis_name="c", subcore_axis_name="s"),
    scratch_types=[
        pltpu.VMEM((8, _BPW), jnp.int32),
        pltpu.VMEM((4, _L, _D), jnp.float32),
        pltpu.VMEM((2, _BPW), jnp.float32),
        pltpu.SemaphoreType.DMA,
    ],
)
def _ttranse_sc(idx_hbm, ent_hbm, rel_hbm,
                out_correct, out_corrupt,
                idx_v, rows_v, out_v, sem):
    wid = lax.axis_index("s") * _NC + lax.axis_index("c")
    base = wid * _BPW

    pltpu.sync_copy(idx_hbm.at[:, pl.ds(base, _BPW)], idx_v)

    tabs = (ent_hbm, rel_hbm, ent_hbm, rel_hbm)

    lane = lax.iota(jnp.int32, _L)
    perms = [lane ^ m for m in (8, 4, 2, 1)]

    def lane_shuffle(v, idx):
        return lax.gather(
            v, idx[:, None],
            lax.GatherDimensionNumbers(
                offset_dims=(), collapsed_slice_dims=(0,),
                start_index_map=(0,)),
            (1,), mode=lax.GatherScatterMode.PROMISE_IN_BOUNDS)

    for b in range(2):
        def wave(w, carry, b=b):
            rid = [idx_v[4 * b + t, pl.ds(w * _L, _L)] for t in range(4)]
            copies = [
                pltpu.async_copy(
                    tabs[t].at[pl.ds(rid[t][j], 1)],
                    rows_v.at[t, pl.ds(j, 1)], sem)
                for t in range(4)
                for j in range(_L)
            ]
            for c in copies:
                c.wait()

            vec = jnp.zeros((_L,), jnp.float32)
            for j in range(_L):
                acc = jnp.zeros((_L,), jnp.float32)
                for c in range(_D // _L):
                    sl_c = pl.ds(c * _L, _L)
                    h = rows_v[0, j, sl_c]
                    r = rows_v[1, j, sl_c]
                    t2 = rows_v[2, j, sl_c]
                    tt = rows_v[3, j, sl_c]
                    v = h + r + tt - t2
                    acc = acc + v * v
                for pm in perms:
                    acc = acc + lane_shuffle(acc, pm)
                onehot = (1 - jnp.minimum(jnp.abs(lane - j), 1)
                          ).astype(jnp.float32)
                vec = vec + acc * onehot
            out_v[b, pl.ds(w * _L, _L)] = vec
            return carry

        lax.fori_loop(0, _W, wave, 0)

    pltpu.sync_copy(out_v.at[0], out_correct.at[pl.ds(base, _BPW)])
    pltpu.sync_copy(out_v.at[1], out_corrupt.at[pl.ds(base, _BPW)])


def kernel(batch, corrupt_batch, entity_emb, relation_emb):
    idx = jnp.concatenate([batch.T, corrupt_batch.T], axis=0)
    correct, corrupt = _ttranse_sc(idx, entity_emb, relation_emb)
    return (correct, corrupt)
